# baseline (speedup 1.0000x reference)
import functools

import jax
import jax.numpy as jnp
from jax.experimental import pallas as pl
from jax.experimental.pallas import tpu as pltpu
from jax.experimental.pallas import tpu_sc as plsc

_LANES = 16


@jax.jit
def kernel(x, token_table, pos_table):
    B, L = x.shape
    V, D = token_table.shape
    N = B * L
    W = 8 * L

    tok_128 = (
        token_table.reshape(V // 32, 4, 8, D)
        .transpose(0, 2, 1, 3)
        .reshape(V // 4, 128)
    )
    tok_128 = jax.lax.optimization_barrier(tok_128)
    tok_lin = tok_128.reshape(V, D)
    xi = x.reshape(N).astype(jnp.int32)
    x_flat = (xi & ~jnp.int32(31)) | ((xi & 7) << 2) | ((xi >> 3) & 3)

    @functools.partial(
        pl.kernel,
        out_type=jax.ShapeDtypeStruct((N, D), jnp.float32),
        mesh=plsc.VectorSubcoreMesh(
            core_axis_name="core", subcore_axis_name="subcore"
        ),
        compiler_params=pltpu.CompilerParams(use_tc_tiling_on_sc=False),
    )
    def sc_embed(tok_hbm, idx_hbm, out_hbm):
        def body(i_vmem, o_vmem):
            pltpu.sync_copy(tok_hbm.at[i_vmem], o_vmem)

        pltpu.emit_pipeline(
            body,
            grid=(N // W,),
            in_specs=[pl.BlockSpec((W,), lambda i: (i,))],
            out_specs=[pl.BlockSpec((W, D), lambda i: (i, 0))],
            core_axis_name=("core", "subcore"),
            dimension_semantics=(pltpu.PARALLEL,),
        )(idx_hbm, out_hbm)

    flat = sc_embed(tok_lin, x_flat)

    G = L // 4
    t2 = flat.reshape(N // 4, 128)
    BB = 256

    pos128 = pos_table.reshape(G, 128)

    def tc_body(in_ref, pos_ref, out_ref):
        v = in_ref[...].reshape(BB, G, 128)
        for g in range(G):
            out_ref[g] = v[:, g, :].T + pos_ref[g][:, None]

    out128 = pl.pallas_call(
        tc_body,
        grid=(B // BB,),
        in_specs=[
            pl.BlockSpec((G * BB, 128), lambda i: (i, 0)),
            pl.BlockSpec((G, 128), lambda i: (0, 0)),
        ],
        out_specs=pl.BlockSpec((G, 128, BB), lambda i: (0, 0, i)),
        out_shape=jax.ShapeDtypeStruct((G, 128, B), jnp.float32),
    )(t2, pos128)
    return out128.reshape(L, D, B).transpose(2, 0, 1)


# speedup vs baseline: 2.0043x; 2.0043x over previous
import functools

import jax
import jax.numpy as jnp
from jax.experimental import pallas as pl
from jax.experimental.pallas import tpu as pltpu
from jax.experimental.pallas import tpu_sc as plsc

_LANES = 16


@jax.jit
def kernel(x, token_table, pos_table):
    B, L = x.shape
    V, D = token_table.shape
    N = B * L
    W = 8 * L

    x_flat = x.reshape(N).astype(jnp.int32)

    tok_t = token_table.T
    CW = 512
    V_main = (V // CW) * CW

    @functools.partial(
        pl.kernel,
        out_type=jax.ShapeDtypeStruct((V // 4, 128), jnp.float32),
        mesh=plsc.VectorSubcoreMesh(
            core_axis_name="core", subcore_axis_name="subcore"
        ),
        compiler_params=pltpu.CompilerParams(
            use_tc_tiling_on_sc=True, needs_layout_passes=False
        ),
    )
    def sc_pack(tokt_hbm, out_hbm):
        def body(in_vmem, o_vmem):
            iota = jax.lax.iota(jnp.int32, _LANES)
            qoff = iota // 4
            colpart = (iota % 4) * D

            @pl.loop(0, CW, step=_LANES)
            def _(v0):
                cols_ld = v0 + iota
                rows_st = v0 // 4 + qoff
                for d0 in range(D):
                    dvec = (d0 + iota) & (D - 1)
                    vals = plsc.load_gather(in_vmem, [dvec, cols_ld])
                    plsc.store_scatter(
                        o_vmem, [rows_st, colpart + dvec], vals
                    )

        pltpu.emit_pipeline(
            body,
            grid=(V_main // CW,),
            in_specs=[pl.BlockSpec((D, CW), lambda i: (0, i))],
            out_specs=[pl.BlockSpec((CW // 4, 128), lambda i: (i, 0))],
            core_axis_name=("core", "subcore"),
            dimension_semantics=(pltpu.PARALLEL,),
        )(tokt_hbm, out_hbm)

    packed_main = sc_pack(tok_t)

    tail_vals = jax.lax.slice(token_table, (V_main, 0), (V, D))
    tok_lin = jax.lax.dynamic_update_slice(
        packed_main, tail_vals.reshape((V - V_main) // 4, 128), (V_main // 4, 0)
    ).reshape(V, D)

    @functools.partial(
        pl.kernel,
        out_type=jax.ShapeDtypeStruct((N, D), jnp.float32),
        mesh=plsc.VectorSubcoreMesh(
            core_axis_name="core", subcore_axis_name="subcore"
        ),
        compiler_params=pltpu.CompilerParams(use_tc_tiling_on_sc=False),
    )
    def sc_embed(tok_hbm, idx_hbm, out_hbm):
        def body(i_vmem, o_vmem):
            pltpu.sync_copy(tok_hbm.at[i_vmem], o_vmem)

        pltpu.emit_pipeline(
            body,
            grid=(N // W,),
            in_specs=[pl.BlockSpec((W,), lambda i: (i,))],
            out_specs=[pl.BlockSpec((W, D), lambda i: (i, 0))],
            core_axis_name=("core", "subcore"),
            dimension_semantics=(pltpu.PARALLEL,),
        )(idx_hbm, out_hbm)

    flat = sc_embed(tok_lin, x_flat)

    G = L // 4
    t2 = flat.reshape(N // 4, 128)
    BB = 256

    pos128 = pos_table.reshape(G, 128)

    def tc_body(in_ref, pos_ref, out_ref):
        v = in_ref[...].reshape(BB, G, 128)
        for g in range(G):
            out_ref[g] = v[:, g, :].T + pos_ref[g][:, None]

    out128 = pl.pallas_call(
        tc_body,
        grid=(B // BB,),
        in_specs=[
            pl.BlockSpec((G * BB, 128), lambda i: (i, 0)),
            pl.BlockSpec((G, 128), lambda i: (0, 0)),
        ],
        out_specs=pl.BlockSpec((G, 128, BB), lambda i: (0, 0, i)),
        out_shape=jax.ShapeDtypeStruct((G, 128, B), jnp.float32),
    )(t2, pos128)
    return out128.reshape(L, D, B).transpose(2, 0, 1)


# speedup vs baseline: 2.0971x; 1.0463x over previous
import functools

import jax
import jax.numpy as jnp
from jax.experimental import pallas as pl
from jax.experimental.pallas import tpu as pltpu
from jax.experimental.pallas import tpu_sc as plsc

_LANES = 16


@jax.jit
def kernel(x, token_table, pos_table):
    B, L = x.shape
    V, D = token_table.shape
    N = B * L
    W = 8 * L

    x_flat = x.reshape(N).astype(jnp.int32)

    tok_t = token_table.T
    CW = 512
    V_main = (V // CW) * CW

    @functools.partial(
        pl.kernel,
        out_type=jax.ShapeDtypeStruct((V // 4, 128), jnp.float32),
        mesh=plsc.VectorSubcoreMesh(
            core_axis_name="core", subcore_axis_name="subcore"
        ),
        compiler_params=pltpu.CompilerParams(
            use_tc_tiling_on_sc=True, needs_layout_passes=False
        ),
    )
    def sc_pack(tokt_hbm, out_hbm):
        def body(in_vmem, o_vmem):
            iota = jax.lax.iota(jnp.int32, _LANES)
            qoff = iota // 4
            colpart = (iota % 4) * D

            @pl.loop(0, CW, step=_LANES, unroll=2)
            def _(v0):
                cols_ld = v0 + iota
                rows_st = v0 // 4 + qoff
                for d0 in range(D):
                    dvec = (d0 + iota) & (D - 1)
                    vals = plsc.load_gather(in_vmem, [dvec, cols_ld])
                    plsc.store_scatter(
                        o_vmem, [rows_st, colpart + dvec], vals
                    )

        pltpu.emit_pipeline(
            body,
            grid=(V_main // CW,),
            in_specs=[pl.BlockSpec((D, CW), lambda i: (0, i))],
            out_specs=[pl.BlockSpec((CW // 4, 128), lambda i: (i, 0))],
            core_axis_name=("core", "subcore"),
            dimension_semantics=(pltpu.PARALLEL,),
        )(tokt_hbm, out_hbm)

    packed_main = sc_pack(tok_t)

    tail_vals = jax.lax.slice(token_table, (V_main, 0), (V, D))
    tok_lin = jax.lax.dynamic_update_slice(
        packed_main, tail_vals.reshape((V - V_main) // 4, 128), (V_main // 4, 0)
    ).reshape(V, D)

    @functools.partial(
        pl.kernel,
        out_type=jax.ShapeDtypeStruct((N, D), jnp.float32),
        mesh=plsc.VectorSubcoreMesh(
            core_axis_name="core", subcore_axis_name="subcore"
        ),
        compiler_params=pltpu.CompilerParams(use_tc_tiling_on_sc=False),
    )
    def sc_embed(tok_hbm, idx_hbm, out_hbm):
        def body(i_vmem, o_vmem):
            pltpu.sync_copy(tok_hbm.at[i_vmem], o_vmem)

        pltpu.emit_pipeline(
            body,
            grid=(N // W,),
            in_specs=[pl.BlockSpec((W,), lambda i: (i,))],
            out_specs=[pl.BlockSpec((W, D), lambda i: (i, 0))],
            core_axis_name=("core", "subcore"),
            dimension_semantics=(pltpu.PARALLEL,),
        )(idx_hbm, out_hbm)

    flat = sc_embed(tok_lin, x_flat)

    G = L // 4
    t2 = flat.reshape(N // 4, 128)
    BB = 256

    pos128 = pos_table.reshape(G, 128)

    def tc_body(in_ref, pos_ref, out_ref):
        v = in_ref[...].reshape(BB, G, 128)
        for g in range(G):
            out_ref[g] = v[:, g, :].T + pos_ref[g][:, None]

    out128 = pl.pallas_call(
        tc_body,
        grid=(B // BB,),
        in_specs=[
            pl.BlockSpec((G * BB, 128), lambda i: (i, 0)),
            pl.BlockSpec((G, 128), lambda i: (0, 0)),
        ],
        out_specs=pl.BlockSpec((G, 128, BB), lambda i: (0, 0, i)),
        out_shape=jax.ShapeDtypeStruct((G, 128, B), jnp.float32),
    )(t2, pos128)
    return out128.reshape(L, D, B).transpose(2, 0, 1)


# speedup vs baseline: 2.1251x; 1.0134x over previous
import functools

import jax
import jax.numpy as jnp
from jax.experimental import pallas as pl
from jax.experimental.pallas import tpu as pltpu
from jax.experimental.pallas import tpu_sc as plsc

_LANES = 16


@jax.jit
def kernel(x, token_table, pos_table):
    B, L = x.shape
    V, D = token_table.shape
    N = B * L
    W = 8 * L

    x_flat = x.reshape(N).astype(jnp.int32)

    tok_t = token_table.T
    CW = 512
    V_main = (V // CW) * CW

    @functools.partial(
        pl.kernel,
        out_type=jax.ShapeDtypeStruct((V // 4, 128), jnp.float32),
        mesh=plsc.VectorSubcoreMesh(
            core_axis_name="core", subcore_axis_name="subcore"
        ),
        compiler_params=pltpu.CompilerParams(
            use_tc_tiling_on_sc=True, needs_layout_passes=False
        ),
    )
    def sc_pack(tokt_hbm, out_hbm):
        def body(in_vmem, o_vmem):
            iota = jax.lax.iota(jnp.int32, _LANES)
            qoff = iota // 4
            colpart = (iota % 4) * D

            @pl.loop(0, CW, step=_LANES, unroll=2)
            def _(v0):
                cols_ld = v0 + iota
                rows_st = v0 // 4 + qoff
                for d0 in range(D):
                    dvec = (d0 + iota) & (D - 1)
                    vals = plsc.load_gather(in_vmem, [dvec, cols_ld])
                    plsc.store_scatter(
                        o_vmem, [rows_st, colpart + dvec], vals
                    )

        pltpu.emit_pipeline(
            body,
            grid=(V_main // CW,),
            in_specs=[pl.BlockSpec((D, CW), lambda i: (0, i))],
            out_specs=[pl.BlockSpec((CW // 4, 128), lambda i: (i, 0))],
            core_axis_name=("core", "subcore"),
            dimension_semantics=(pltpu.PARALLEL,),
        )(tokt_hbm, out_hbm)

    packed_main = sc_pack(tok_t)

    tail_vals = jax.lax.slice(token_table, (V_main, 0), (V, D))
    tok_lin = jax.lax.dynamic_update_slice(
        packed_main, tail_vals.reshape((V - V_main) // 4, 128), (V_main // 4, 0)
    ).reshape(V, D)

    @functools.partial(
        pl.kernel,
        out_type=jax.ShapeDtypeStruct((N // 2, D), jnp.float32),
        mesh=plsc.VectorSubcoreMesh(
            core_axis_name="core", subcore_axis_name="subcore"
        ),
        compiler_params=pltpu.CompilerParams(use_tc_tiling_on_sc=False),
    )
    def sc_embed(tok_hbm, idx_hbm, out_hbm):
        def body(i_vmem, o_vmem):
            pltpu.sync_copy(tok_hbm.at[i_vmem], o_vmem)

        pltpu.emit_pipeline(
            body,
            grid=(N // 2 // W,),
            in_specs=[pl.BlockSpec((W,), lambda i: (i,))],
            out_specs=[pl.BlockSpec((W, D), lambda i: (i, 0))],
            core_axis_name=("core", "subcore"),
            dimension_semantics=(pltpu.PARALLEL,),
        )(idx_hbm, out_hbm)

    flat_a = sc_embed(tok_lin, x_flat[: N // 2])
    flat_b = sc_embed(tok_lin, x_flat[N // 2 :])

    G = L // 4
    BB = 256
    HC = B // 2 // BB

    pos128 = pos_table.reshape(G, 128)

    def tc_body(in_ref, pos_ref, out_ref):
        v = in_ref[...].reshape(BB, G, 128)
        for g in range(G):
            out_ref[g] = v[:, g, :].T + pos_ref[g][:, None]

    def tc_body_b(in_ref, pos_ref, prev_ref, out_ref):
        tc_body(in_ref, pos_ref, out_ref)

    out_a = pl.pallas_call(
        tc_body,
        grid=(HC,),
        in_specs=[
            pl.BlockSpec((G * BB, 128), lambda i: (i, 0)),
            pl.BlockSpec((G, 128), lambda i: (0, 0)),
        ],
        out_specs=pl.BlockSpec((G, 128, BB), lambda i: (0, 0, i)),
        out_shape=jax.ShapeDtypeStruct((G, 128, B), jnp.float32),
    )(flat_a.reshape(N // 8, 128), pos128)

    out128 = pl.pallas_call(
        tc_body_b,
        grid=(HC,),
        in_specs=[
            pl.BlockSpec((G * BB, 128), lambda i: (i, 0)),
            pl.BlockSpec((G, 128), lambda i: (0, 0)),
            pl.BlockSpec((1, 128, BB), lambda i: (0, 0, 0)),
        ],
        out_specs=pl.BlockSpec((G, 128, BB), lambda i: (0, 0, i + HC)),
        out_shape=jax.ShapeDtypeStruct((G, 128, B), jnp.float32),
        input_output_aliases={2: 0},
    )(flat_b.reshape(N // 8, 128), pos128, out_a)
    return out128.reshape(L, D, B).transpose(2, 0, 1)
